# Initial kernel scaffold; baseline (speedup 1.0000x reference)
import jax
import jax.numpy as jnp
from jax.experimental import pallas as pl


def kernel(x, edge_index, edge_weight, W1_l, W1_r, b1, W2_l, W2_r, b2, W3_l, W3_r, b3):
    raise NotImplementedError("write your pallas kernel here")


# speedup vs baseline: 4.6920x; 4.6920x over previous
import functools

import jax
import jax.numpy as jnp
from jax import lax
from jax.experimental import pallas as pl
from jax.experimental.pallas import tpu as pltpu
from jax.experimental.pallas import tpu_sc as plsc

N = 10000
E = 320000
D = 128
HD = D // 2

NC = 2
NS = 16
L = 16

EPT = E // NS
CHUNK = 80
NCHUNKS = EPT // CHUNK
NBUF = 3
NP_ = 10240
RPT = NP_ // NS
ZROWS = 64
DEGW = 8

BN = 400

_GATHER_DNUMS = lax.GatherDimensionNumbers(
    offset_dims=(), collapsed_slice_dims=(0,), start_index_map=(0,))


def _lane_bcast(vec, lane):
    idx = jnp.full((L, 1), lane, jnp.int32)
    return lax.gather(vec, idx, _GATHER_DNUMS, (1,),
                      mode=lax.GatherScatterMode.PROMISE_IN_BOUNDS)


def _make_spmm(with_deg):
    mesh = plsc.VectorSubcoreMesh(core_axis_name="c", subcore_axis_name="s")
    out_type = [jax.ShapeDtypeStruct((NC, NP_, HD), jnp.float32)]
    scratch = [
        pltpu.VMEM_SHARED((NP_, HD), jnp.float32),
        pltpu.VMEM((ZROWS, HD), jnp.float32),
        pltpu.VMEM((NCHUNKS, CHUNK), jnp.int32),
        pltpu.VMEM((NCHUNKS, CHUNK), jnp.int32),
        pltpu.VMEM((NCHUNKS, CHUNK), jnp.float32),
    ]
    scratch += [pltpu.VMEM((CHUNK, HD), jnp.float32) for _ in range(NBUF)]
    scratch += [pltpu.SemaphoreType.DMA for _ in range(2 * NBUF)]
    if with_deg:
        out_type.append(jax.ShapeDtypeStruct((NC, NP_, DEGW), jnp.float32))
        scratch += [
            pltpu.VMEM_SHARED((NP_, DEGW), jnp.float32),
            pltpu.VMEM((RPT // 5, DEGW), jnp.float32),
            pltpu.VMEM((CHUNK, DEGW), jnp.float32),
        ]
        scratch += [pltpu.SemaphoreType.DMA for _ in range(NBUF)]

    @functools.partial(
        pl.kernel, out_type=tuple(out_type), mesh=mesh,
        scratch_types=tuple(scratch),
        compiler_params=pltpu.CompilerParams(use_tc_tiling_on_sc=False))
    def spmm(src_hbm, dst_hbm, ew_hbm, h_hbm, *refs):
        if with_deg:
            (acc_out, deg_out, acc, zbuf, srcall, dstall, ewall,
             r0, r1, r2, g0, g1, g2, s0, s1, s2,
             accd, zbufd, onesb, d0, d1, d2) = refs
            dsems = [d0, d1, d2]
        else:
            (acc_out, acc, zbuf, srcall, dstall, ewall,
             r0, r1, r2, g0, g1, g2, s0, s1, s2) = refs
        rows = [r0, r1, r2]
        gsems = [g0, g1, g2]
        ssems = [s0, s1, s2]

        cid = lax.axis_index("c")
        sid = lax.axis_index("s")
        htab = h_hbm.at[cid]

        pltpu.sync_copy(src_hbm.at[sid], srcall)
        pltpu.sync_copy(dst_hbm.at[sid], dstall)
        pltpu.sync_copy(ew_hbm.at[sid], ewall)

        def issue_gather(tc, b):
            pltpu.async_copy(htab.at[srcall.at[tc]], rows[b], gsems[b])

        def wait_gather(b):
            pltpu.make_async_copy(htab.at[srcall.at[0]], rows[b],
                                  gsems[b]).wait()

        def issue_scatter(tc, b):
            pltpu.async_copy(rows[b], acc.at[dstall.at[tc]], ssems[b],
                             add=True)
            if with_deg:
                pltpu.async_copy(onesb, accd.at[dstall.at[tc]], dsems[b],
                                 add=True)

        def wait_scatter(b):
            pltpu.make_async_copy(rows[b], acc.at[dstall.at[0]],
                                  ssems[b]).wait()
            if with_deg:
                pltpu.make_async_copy(onesb, accd.at[dstall.at[0]],
                                      dsems[b]).wait()

        issue_gather(0, 0)
        issue_gather(1, 1)

        def zb(r, _):
            for j in range(HD // L):
                zbuf[r, pl.ds(j * L, L)] = jnp.zeros((L,), jnp.float32)
            return 0
        lax.fori_loop(0, ZROWS, zb, 0)
        for k in range(RPT // ZROWS):
            pltpu.sync_copy(zbuf, acc.at[pl.ds(sid * RPT + k * ZROWS, ZROWS)])
        if with_deg:
            def zbd(r, _):
                zbufd[r, :] = jnp.zeros((DEGW,), jnp.float32)
                return 0
            lax.fori_loop(0, RPT // 5, zbd, 0)
            for k in range(5):
                pltpu.sync_copy(
                    zbufd, accd.at[pl.ds(sid * RPT + k * (RPT // 5),
                                         RPT // 5)])

            def ob(g, _):
                onesb[g, :] = jnp.ones((DEGW,), jnp.float32)
                return 0
            lax.fori_loop(0, CHUNK, ob, 0)
        plsc.subcore_barrier()

        def process(tc, b, bn):
            wait_gather(b)

            def group_body(g, _):
                ewg = ewall[tc, pl.ds(g * L, L)]
                for e16 in range(L):
                    wb = _lane_bcast(ewg, e16)
                    e = g * L + e16
                    for j in range(HD // L):
                        rows[b][e, pl.ds(j * L, L)] = (
                            rows[b][e, pl.ds(j * L, L)] * wb)
                return 0
            lax.fori_loop(0, CHUNK // L, group_body, 0)

            issue_scatter(tc, b)

            @pl.when(tc >= 1)
            def _():
                wait_scatter(bn)

            @pl.when(tc + 2 < NCHUNKS)
            def _():
                issue_gather(tc + 2, bn)

        def tri_body(i, _):
            for p in range(NBUF):
                tc = NBUF * i + p

                @pl.when(tc < NCHUNKS)
                def _():
                    process(tc, p, (p + 2) % NBUF)
            return 0
        lax.fori_loop(0, (NCHUNKS + NBUF - 1) // NBUF, tri_body, 0)
        wait_scatter((NCHUNKS - 1) % NBUF)
        plsc.subcore_barrier()

        pltpu.sync_copy(acc.at[pl.ds(sid * RPT, RPT)],
                        acc_out.at[cid, pl.ds(sid * RPT, RPT)])
        if with_deg:
            pltpu.sync_copy(accd.at[pl.ds(sid * RPT, RPT)],
                            deg_out.at[cid, pl.ds(sid * RPT, RPT)])

    return spmm


def _dense_body(last, acc_ref, deg_ref, h_ref, wl_ref, wr_ref, b_ref, o_ref):
    deg = jnp.clip(deg_ref[0, :, :1], 1.0, None)
    a0 = acc_ref[0] / deg
    a1 = acc_ref[1] / deg
    y = jnp.dot(a0, wl_ref[:HD, :], preferred_element_type=jnp.float32)
    y = y + jnp.dot(a1, wl_ref[HD:, :], preferred_element_type=jnp.float32)
    y = y + jnp.dot(h_ref[0], wr_ref[:HD, :],
                    preferred_element_type=jnp.float32)
    y = y + jnp.dot(h_ref[1], wr_ref[HD:, :],
                    preferred_element_type=jnp.float32)
    y = y + b_ref[...]
    if last:
        o_ref[...] = y
    else:
        y = jnp.maximum(y, 0.0)
        o_ref[0] = y[:, :HD]
        o_ref[1] = y[:, HD:]


def _make_dense(last):
    if last:
        out_shape = jax.ShapeDtypeStruct((N, D), jnp.float32)
        out_specs = pl.BlockSpec((BN, D), lambda i: (i, 0))
    else:
        out_shape = jax.ShapeDtypeStruct((NC, N, HD), jnp.float32)
        out_specs = pl.BlockSpec((NC, BN, HD), lambda i: (0, i, 0))
    return pl.pallas_call(
        functools.partial(_dense_body, last),
        grid=(N // BN,),
        in_specs=[
            pl.BlockSpec((NC, BN, HD), lambda i: (0, i, 0)),
            pl.BlockSpec((NC, BN, DEGW), lambda i: (0, i, 0)),
            pl.BlockSpec((NC, BN, HD), lambda i: (0, i, 0)),
            pl.BlockSpec((D, D), lambda i: (0, 0)),
            pl.BlockSpec((D, D), lambda i: (0, 0)),
            pl.BlockSpec((1, D), lambda i: (0, 0)),
        ],
        out_specs=out_specs,
        out_shape=out_shape,
    )


_spmm_deg = _make_spmm(True)
_spmm_nodeg = _make_spmm(False)
_dense_mid = _make_dense(False)
_dense_last = _make_dense(True)


def kernel(x, edge_index, edge_weight, W1_l, W1_r, b1, W2_l, W2_r, b2,
           W3_l, W3_r, b3):
    src = edge_index[0].astype(jnp.int32).reshape(NS, NCHUNKS, CHUNK)
    dst = edge_index[1].astype(jnp.int32).reshape(NS, NCHUNKS, CHUNK)
    ew = edge_weight.astype(jnp.float32).reshape(NS, NCHUNKS, CHUNK)
    xs = jnp.stack([x[:, :HD], x[:, HD:]])

    acc, deg = _spmm_deg(src, dst, ew, xs)
    hs = _dense_mid(acc, deg, xs, W1_l, W1_r, b1.reshape(1, D))
    (acc,) = _spmm_nodeg(src, dst, ew, hs)
    hs = _dense_mid(acc, deg, hs, W2_l, W2_r, b2.reshape(1, D))
    (acc,) = _spmm_nodeg(src, dst, ew, hs)
    out = _dense_last(acc, deg, hs, W3_l, W3_r, b3.reshape(1, D))
    return out


# speedup vs baseline: 10.8310x; 2.3084x over previous
import functools

import jax
import jax.numpy as jnp
import numpy as np
from jax import lax
from jax.experimental import pallas as pl
from jax.experimental.pallas import tpu as pltpu
from jax.experimental.pallas import tpu_sc as plsc

N = 10000
E = 320000
D = 128
HD = D // 2
HD2 = HD // 2

NC = 2
NS = 16
L = 16

EPT = E // NS
CHUNK = 80
NCHUNKS = EPT // CHUNK
NBUF = 2
NP_ = 10240
RPT = NP_ // NS
ZROWS = 32
DEGW = 8

BN = 400

_GATHER_DNUMS = lax.GatherDimensionNumbers(
    offset_dims=(), collapsed_slice_dims=(0,), start_index_map=(0,))

_PHALF = np.concatenate([np.arange(0, 32, 2), np.arange(1, 32, 2),
                         32 + np.arange(0, 32, 2), 33 + np.arange(0, 32, 2)])
_PERM = np.concatenate([_PHALF, HD + _PHALF])


def _lane_bcast(vec, lane):
    idx = jnp.full((L, 1), lane, jnp.int32)
    return lax.gather(vec, idx, _GATHER_DNUMS, (1,),
                      mode=lax.GatherScatterMode.PROMISE_IN_BOUNDS)


NIB = 6
NRB = 3
NSB = 2


def _make_spmm(with_deg):
    mesh = plsc.VectorSubcoreMesh(core_axis_name="c", subcore_axis_name="s")
    out_type = [jax.ShapeDtypeStruct((NC, NP_, HD), jnp.float32)]
    scratch = [
        pltpu.VMEM_SHARED((NP_, HD), jnp.float32),
        pltpu.VMEM((ZROWS, HD), jnp.float32),
    ]
    scratch += [pltpu.VMEM((CHUNK,), jnp.int32) for _ in range(NIB)]
    scratch += [pltpu.VMEM((CHUNK,), jnp.int32) for _ in range(NIB)]
    scratch += [pltpu.VMEM((CHUNK,), jnp.float32) for _ in range(NIB)]
    scratch += [pltpu.VMEM((CHUNK, HD), jnp.float32) for _ in range(NRB)]
    scratch += [pltpu.VMEM((CHUNK, HD), jnp.float32) for _ in range(NSB)]
    scratch += [pltpu.SemaphoreType.DMA for _ in range(NIB + NRB + NSB)]
    if with_deg:
        out_type.append(jax.ShapeDtypeStruct((NC, NP_, DEGW), jnp.float32))
        scratch += [
            pltpu.VMEM_SHARED((NP_, DEGW), jnp.float32),
            pltpu.VMEM((RPT // 5, DEGW), jnp.float32),
            pltpu.VMEM((CHUNK, DEGW), jnp.float32),
        ]
        scratch += [pltpu.SemaphoreType.DMA for _ in range(NSB)]

    @functools.partial(
        pl.kernel, out_type=tuple(out_type), mesh=mesh,
        scratch_types=tuple(scratch),
        compiler_params=pltpu.CompilerParams(use_tc_tiling_on_sc=False))
    def spmm(src_hbm, dst_hbm, ew_hbm, h_hbm, *refs):
        refs = list(refs)
        acc_out = refs.pop(0)
        deg_out = refs.pop(0) if with_deg else None
        acc = refs.pop(0)
        zbuf = refs.pop(0)
        srcb = [refs.pop(0) for _ in range(NIB)]
        dstb = [refs.pop(0) for _ in range(NIB)]
        ewb = [refs.pop(0) for _ in range(NIB)]
        rows = [refs.pop(0) for _ in range(NRB)]
        sbuf = [refs.pop(0) for _ in range(NSB)]
        isems = [refs.pop(0) for _ in range(NIB)]
        gsems = [refs.pop(0) for _ in range(NRB)]
        ssems = [refs.pop(0) for _ in range(NSB)]
        if with_deg:
            accd, zbufd, onesb = refs.pop(0), refs.pop(0), refs.pop(0)
            dsems = [refs.pop(0) for _ in range(NSB)]

        cid = lax.axis_index("c")
        sid = lax.axis_index("s")
        htab = h_hbm.at[cid]

        def issue_stage(tc, k):
            pltpu.async_copy(src_hbm.at[sid, tc], srcb[k], isems[k])
            pltpu.async_copy(dst_hbm.at[sid, tc], dstb[k], isems[k])
            pltpu.async_copy(ew_hbm.at[sid, tc], ewb[k], isems[k])

        def wait_stage(k):
            pltpu.make_async_copy(src_hbm.at[sid, 0], srcb[k], isems[k]).wait()
            pltpu.make_async_copy(dst_hbm.at[sid, 0], dstb[k], isems[k]).wait()
            pltpu.make_async_copy(ew_hbm.at[sid, 0], ewb[k], isems[k]).wait()

        def issue_gather(k, r):
            pltpu.async_copy(htab.at[srcb[k]], rows[r], gsems[r])

        def wait_gather(r):
            pltpu.make_async_copy(htab.at[srcb[0]], rows[r], gsems[r]).wait()

        def issue_scatter(k, b):
            pltpu.async_copy(sbuf[b], acc.at[dstb[k]], ssems[b], add=True)
            if with_deg:
                pltpu.async_copy(onesb, accd.at[dstb[k]], dsems[b], add=True)

        def wait_scatter(b):
            pltpu.make_async_copy(sbuf[b], acc.at[dstb[0]], ssems[b]).wait()
            if with_deg:
                pltpu.make_async_copy(onesb, accd.at[dstb[0]],
                                      dsems[b]).wait()

        for k in range(4):
            issue_stage(k, k)
        wait_stage(0)
        issue_gather(0, 0)
        wait_stage(1)
        issue_gather(1, 1)

        def zb(r, _):
            for j in range(HD // L):
                zbuf[r, pl.ds(j * L, L)] = jnp.zeros((L,), jnp.float32)
            return 0
        lax.fori_loop(0, ZROWS, zb, 0)
        for k in range(RPT // ZROWS):
            pltpu.sync_copy(zbuf, acc.at[pl.ds(sid * RPT + k * ZROWS, ZROWS)])
        if with_deg:
            def zbd(r, _):
                zbufd[r, :] = jnp.zeros((DEGW,), jnp.float32)
                return 0
            lax.fori_loop(0, RPT // 5, zbd, 0)
            for k in range(5):
                pltpu.sync_copy(
                    zbufd, accd.at[pl.ds(sid * RPT + k * (RPT // 5),
                                         RPT // 5)])

            def ob(g, _):
                onesb[g, :] = jnp.ones((DEGW,), jnp.float32)
                return 0
            lax.fori_loop(0, CHUNK, ob, 0)
        plsc.subcore_barrier()

        def process(tc, p):
            ib, rb, sb = p % NIB, p % NRB, p % NSB

            @pl.when(tc >= NSB)
            def _():
                wait_scatter(sb)

            @pl.when(tc + 4 < NCHUNKS)
            def _():
                issue_stage(tc + 4, (p + 4) % NIB)

            @pl.when(tc + 2 < NCHUNKS)
            def _():
                wait_stage((p + 2) % NIB)
                issue_gather((p + 2) % NIB, (p + 2) % NRB)

            wait_gather(rb)
            rbuf, obuf = rows[rb], sbuf[sb]
            eww = ewb[ib]

            def group_body(g, _):
                ewg = eww[pl.ds(g * L, L)]
                rsl = rbuf.at[pl.ds(g * L, L)]
                osl = obuf.at[pl.ds(g * L, L)]
                for e16 in range(L):
                    wb = _lane_bcast(ewg, e16)
                    for j in range(HD // L):
                        osl[e16, pl.ds(j * L, L)] = (
                            rsl[e16, pl.ds(j * L, L)] * wb)
                return 0
            lax.fori_loop(0, CHUNK // L, group_body, 0)

            issue_scatter(ib, sb)

        def hex_body(i, _):
            for p in range(NIB):
                tc = NIB * i + p

                @pl.when(tc < NCHUNKS)
                def _():
                    process(tc, p)
            return 0
        lax.fori_loop(0, (NCHUNKS + NIB - 1) // NIB, hex_body, 0)
        for b in range(NSB):
            wait_scatter(b)
        plsc.subcore_barrier()

        pltpu.sync_copy(acc.at[pl.ds(sid * RPT, RPT)],
                        acc_out.at[cid, pl.ds(sid * RPT, RPT)])
        if with_deg:
            pltpu.sync_copy(accd.at[pl.ds(sid * RPT, RPT)],
                            deg_out.at[cid, pl.ds(sid * RPT, RPT)])

    return spmm


def _dense_body(last, acc_ref, deg_ref, h_ref, wl_ref, wr_ref, b_ref, *o_refs):
    deg = jnp.clip(deg_ref[0, :, :1], 1.0, None)
    a0 = acc_ref[0] / deg
    a1 = acc_ref[1] / deg
    y = jnp.dot(a0, wl_ref[:HD, :], preferred_element_type=jnp.float32)
    y = y + jnp.dot(a1, wl_ref[HD:, :], preferred_element_type=jnp.float32)
    y = y + jnp.dot(h_ref[0], wr_ref[:HD, :],
                    preferred_element_type=jnp.float32)
    y = y + jnp.dot(h_ref[1], wr_ref[HD:, :],
                    preferred_element_type=jnp.float32)
    y = y + b_ref[...]
    if last:
        o_refs[0][...] = y
    else:
        y = jnp.maximum(y, 0.0)
        o_refs[0][0] = y[:, :HD]
        o_refs[0][1] = y[:, HD:]


def _make_dense(last):
    if last:
        out_shape = jax.ShapeDtypeStruct((N, D), jnp.float32)
        out_specs = pl.BlockSpec((BN, D), lambda i: (i, 0))
    else:
        out_shape = jax.ShapeDtypeStruct((NC, N, HD), jnp.float32)
        out_specs = pl.BlockSpec((NC, BN, HD), lambda i: (0, i, 0))
    return pl.pallas_call(
        functools.partial(_dense_body, last),
        grid=(N // BN,),
        in_specs=[
            pl.BlockSpec((NC, BN, HD), lambda i: (0, i, 0)),
            pl.BlockSpec((NC, BN, DEGW), lambda i: (0, i, 0)),
            pl.BlockSpec((NC, BN, HD), lambda i: (0, i, 0)),
            pl.BlockSpec((D, D), lambda i: (0, 0)),
            pl.BlockSpec((D, D), lambda i: (0, 0)),
            pl.BlockSpec((1, D), lambda i: (0, 0)),
        ],
        out_specs=out_specs,
        out_shape=out_shape,
    )


_spmm_deg = _make_spmm(True)
_spmm_nodeg = _make_spmm(False)
_dense_mid = _make_dense(False)
_dense_last = _make_dense(True)


def kernel(x, edge_index, edge_weight, W1_l, W1_r, b1, W2_l, W2_r, b2,
           W3_l, W3_r, b3):
    src = edge_index[0].astype(jnp.int32).reshape(NS, NCHUNKS, CHUNK)
    dst = edge_index[1].astype(jnp.int32).reshape(NS, NCHUNKS, CHUNK)
    ew = edge_weight.astype(jnp.float32).reshape(NS, NCHUNKS, CHUNK)
    xs = jnp.stack([x[:, :HD], x[:, HD:]])
    acc, deg = _spmm_deg(src, dst, ew, xs)
    hs = _dense_mid(acc, deg, xs, W1_l, W1_r, b1.reshape(1, D))
    (acc,) = _spmm_nodeg(src, dst, ew, hs)
    hs = _dense_mid(acc, deg, hs, W2_l, W2_r, b2.reshape(1, D))
    (acc,) = _spmm_nodeg(src, dst, ew, hs)
    out = _dense_last(acc, deg, hs, W3_l, W3_r, b3.reshape(1, D))
    return out


# speedup vs baseline: 10.8317x; 1.0001x over previous
import functools

import jax
import jax.numpy as jnp
from jax import lax
from jax.experimental import pallas as pl
from jax.experimental.pallas import tpu as pltpu
from jax.experimental.pallas import tpu_sc as plsc

N = 10000
E = 320000
D = 128
HD = D // 2

NC = 2
NS = 16
L = 16

EPT = E // NS
CHUNK = 80
NCHUNKS = EPT // CHUNK
NP_ = 10240
RPT = NP_ // NS
ZROWS = 32
DEGW = 8

BN = 400

_GATHER_DNUMS = lax.GatherDimensionNumbers(
    offset_dims=(), collapsed_slice_dims=(0,), start_index_map=(0,))

def _lane_bcast(vec, lane):
    idx = jnp.full((L, 1), lane, jnp.int32)
    return lax.gather(vec, idx, _GATHER_DNUMS, (1,),
                      mode=lax.GatherScatterMode.PROMISE_IN_BOUNDS)


NIB = 6
NRB = 3
NSB = 2


def _make_spmm(with_deg):
    mesh = plsc.VectorSubcoreMesh(core_axis_name="c", subcore_axis_name="s")
    out_type = [jax.ShapeDtypeStruct((NC, NP_, HD), jnp.float32)]
    scratch = [
        pltpu.VMEM_SHARED((NP_, HD), jnp.float32),
        pltpu.VMEM((ZROWS, HD), jnp.float32),
    ]
    scratch += [pltpu.VMEM((CHUNK,), jnp.int32) for _ in range(NIB)]
    scratch += [pltpu.VMEM((CHUNK,), jnp.int32) for _ in range(NIB)]
    scratch += [pltpu.VMEM((CHUNK,), jnp.float32) for _ in range(NIB)]
    scratch += [pltpu.VMEM((CHUNK, HD), jnp.float32) for _ in range(NRB)]
    scratch += [pltpu.VMEM((CHUNK, HD), jnp.float32) for _ in range(NSB)]
    scratch += [pltpu.SemaphoreType.DMA for _ in range(NIB + NRB + NSB)]
    if with_deg:
        out_type.append(jax.ShapeDtypeStruct((NC, NP_, DEGW), jnp.float32))
        scratch += [
            pltpu.VMEM_SHARED((NP_, DEGW), jnp.float32),
            pltpu.VMEM((RPT // 5, DEGW), jnp.float32),
            pltpu.VMEM((CHUNK, DEGW), jnp.float32),
        ]
        scratch += [pltpu.SemaphoreType.DMA for _ in range(NSB)]

    @functools.partial(
        pl.kernel, out_type=tuple(out_type), mesh=mesh,
        scratch_types=tuple(scratch),
        compiler_params=pltpu.CompilerParams(use_tc_tiling_on_sc=False))
    def spmm(src_hbm, dst_hbm, ew_hbm, h_hbm, *refs):
        refs = list(refs)
        acc_out = refs.pop(0)
        deg_out = refs.pop(0) if with_deg else None
        acc = refs.pop(0)
        zbuf = refs.pop(0)
        srcb = [refs.pop(0) for _ in range(NIB)]
        dstb = [refs.pop(0) for _ in range(NIB)]
        ewb = [refs.pop(0) for _ in range(NIB)]
        rows = [refs.pop(0) for _ in range(NRB)]
        sbuf = [refs.pop(0) for _ in range(NSB)]
        isems = [refs.pop(0) for _ in range(NIB)]
        gsems = [refs.pop(0) for _ in range(NRB)]
        ssems = [refs.pop(0) for _ in range(NSB)]
        if with_deg:
            accd, zbufd, onesb = refs.pop(0), refs.pop(0), refs.pop(0)
            dsems = [refs.pop(0) for _ in range(NSB)]

        cid = lax.axis_index("c")
        sid = lax.axis_index("s")
        htab = h_hbm.at[cid]

        def issue_stage(tc, k):
            pltpu.async_copy(src_hbm.at[sid, tc], srcb[k], isems[k])
            pltpu.async_copy(dst_hbm.at[sid, tc], dstb[k], isems[k])
            pltpu.async_copy(ew_hbm.at[sid, tc], ewb[k], isems[k])

        def wait_stage(k):
            pltpu.make_async_copy(src_hbm.at[sid, 0], srcb[k], isems[k]).wait()
            pltpu.make_async_copy(dst_hbm.at[sid, 0], dstb[k], isems[k]).wait()
            pltpu.make_async_copy(ew_hbm.at[sid, 0], ewb[k], isems[k]).wait()

        def issue_gather(k, r):
            pltpu.async_copy(htab.at[srcb[k]], rows[r], gsems[r])

        def wait_gather(r):
            pltpu.make_async_copy(htab.at[srcb[0]], rows[r], gsems[r]).wait()

        def issue_scatter(k, b):
            pltpu.async_copy(sbuf[b], acc.at[dstb[k]], ssems[b], add=True)
            if with_deg:
                pltpu.async_copy(onesb, accd.at[dstb[k]], dsems[b], add=True)

        def wait_scatter(b):
            pltpu.make_async_copy(sbuf[b], acc.at[dstb[0]], ssems[b]).wait()
            if with_deg:
                pltpu.make_async_copy(onesb, accd.at[dstb[0]],
                                      dsems[b]).wait()

        for k in range(4):
            issue_stage(k, k)
        wait_stage(0)
        issue_gather(0, 0)
        wait_stage(1)
        issue_gather(1, 1)

        def zb(r, _):
            for j in range(HD // L):
                zbuf[r, pl.ds(j * L, L)] = jnp.zeros((L,), jnp.float32)
            return 0
        lax.fori_loop(0, ZROWS, zb, 0)
        for k in range(RPT // ZROWS):
            pltpu.sync_copy(zbuf, acc.at[pl.ds(sid * RPT + k * ZROWS, ZROWS)])
        if with_deg:
            def zbd(r, _):
                zbufd[r, :] = jnp.zeros((DEGW,), jnp.float32)
                return 0
            lax.fori_loop(0, RPT // 5, zbd, 0)
            for k in range(5):
                pltpu.sync_copy(
                    zbufd, accd.at[pl.ds(sid * RPT + k * (RPT // 5),
                                         RPT // 5)])

            def ob(g, _):
                onesb[g, :] = jnp.ones((DEGW,), jnp.float32)
                return 0
            lax.fori_loop(0, CHUNK, ob, 0)
        plsc.subcore_barrier()

        def process(tc, p):
            ib, rb, sb = p % NIB, p % NRB, p % NSB

            @pl.when(tc >= NSB)
            def _():
                wait_scatter(sb)

            @pl.when(tc + 4 < NCHUNKS)
            def _():
                issue_stage(tc + 4, (p + 4) % NIB)

            @pl.when(tc + 2 < NCHUNKS)
            def _():
                wait_stage((p + 2) % NIB)
                issue_gather((p + 2) % NIB, (p + 2) % NRB)

            wait_gather(rb)
            rbuf, obuf = rows[rb], sbuf[sb]
            eww = ewb[ib]

            def group_body(g, _):
                ewg = eww[pl.ds(g * L, L)]
                rsl = rbuf.at[pl.ds(g * L, L)]
                osl = obuf.at[pl.ds(g * L, L)]
                for e16 in range(L):
                    wb = _lane_bcast(ewg, e16)
                    for j in range(HD // L):
                        osl[e16, pl.ds(j * L, L)] = (
                            rsl[e16, pl.ds(j * L, L)] * wb)
                return 0
            lax.fori_loop(0, CHUNK // L, group_body, 0)

            issue_scatter(ib, sb)

        def hex_body(i, _):
            for p in range(NIB):
                tc = NIB * i + p

                @pl.when(tc < NCHUNKS)
                def _():
                    process(tc, p)
            return 0
        lax.fori_loop(0, (NCHUNKS + NIB - 1) // NIB, hex_body, 0)
        for b in range(NSB):
            wait_scatter(b)
        plsc.subcore_barrier()

        pltpu.sync_copy(acc.at[pl.ds(sid * RPT, RPT)],
                        acc_out.at[cid, pl.ds(sid * RPT, RPT)])
        if with_deg:
            pltpu.sync_copy(accd.at[pl.ds(sid * RPT, RPT)],
                            deg_out.at[cid, pl.ds(sid * RPT, RPT)])

    return spmm


def _dense_body(last, acc_ref, deg_ref, h_ref, wl_ref, wr_ref, b_ref, *o_refs):
    deg = jnp.clip(deg_ref[0, :, :1], 1.0, None)
    a0 = acc_ref[0] / deg
    a1 = acc_ref[1] / deg
    y = jnp.dot(a0, wl_ref[:HD, :], preferred_element_type=jnp.float32)
    y = y + jnp.dot(a1, wl_ref[HD:, :], preferred_element_type=jnp.float32)
    y = y + jnp.dot(h_ref[0], wr_ref[:HD, :],
                    preferred_element_type=jnp.float32)
    y = y + jnp.dot(h_ref[1], wr_ref[HD:, :],
                    preferred_element_type=jnp.float32)
    y = y + b_ref[...]
    if last:
        o_refs[0][...] = y
    else:
        y = jnp.maximum(y, 0.0)
        o_refs[0][0] = y[:, :HD]
        o_refs[0][1] = y[:, HD:]


def _make_dense(last):
    if last:
        out_shape = jax.ShapeDtypeStruct((N, D), jnp.float32)
        out_specs = pl.BlockSpec((BN, D), lambda i: (i, 0))
    else:
        out_shape = jax.ShapeDtypeStruct((NC, N, HD), jnp.float32)
        out_specs = pl.BlockSpec((NC, BN, HD), lambda i: (0, i, 0))
    return pl.pallas_call(
        functools.partial(_dense_body, last),
        grid=(N // BN,),
        in_specs=[
            pl.BlockSpec((NC, BN, HD), lambda i: (0, i, 0)),
            pl.BlockSpec((NC, BN, DEGW), lambda i: (0, i, 0)),
            pl.BlockSpec((NC, BN, HD), lambda i: (0, i, 0)),
            pl.BlockSpec((D, D), lambda i: (0, 0)),
            pl.BlockSpec((D, D), lambda i: (0, 0)),
            pl.BlockSpec((1, D), lambda i: (0, 0)),
        ],
        out_specs=out_specs,
        out_shape=out_shape,
    )


_spmm_deg = _make_spmm(True)
_spmm_nodeg = _make_spmm(False)
_dense_mid = _make_dense(False)
_dense_last = _make_dense(True)


def kernel(x, edge_index, edge_weight, W1_l, W1_r, b1, W2_l, W2_r, b2,
           W3_l, W3_r, b3):
    src = edge_index[0].astype(jnp.int32).reshape(NS, NCHUNKS, CHUNK)
    dst = edge_index[1].astype(jnp.int32).reshape(NS, NCHUNKS, CHUNK)
    ew = edge_weight.astype(jnp.float32).reshape(NS, NCHUNKS, CHUNK)
    xs = jnp.stack([x[:, :HD], x[:, HD:]])
    acc, deg = _spmm_deg(src, dst, ew, xs)
    hs = _dense_mid(acc, deg, xs, W1_l, W1_r, b1.reshape(1, D))
    (acc,) = _spmm_nodeg(src, dst, ew, hs)
    hs = _dense_mid(acc, deg, hs, W2_l, W2_r, b2.reshape(1, D))
    (acc,) = _spmm_nodeg(src, dst, ew, hs)
    out = _dense_last(acc, deg, hs, W3_l, W3_r, b3.reshape(1, D))
    return out
